# baseline (speedup 1.0000x reference)
import jax
import jax.numpy as jnp
from jax.experimental import pallas as pl
from jax.experimental.pallas import tpu as pltpu
from jax.experimental.pallas import tpu_sc as plsc

_R = 16
_SEQP = 56


def kernel(input_ids, embedding_table):
    batch, seq = input_ids.shape
    dim = embedding_table.shape[1]
    ids_pad = jnp.pad(input_ids, ((0, 0), (0, 128 - seq)))

    mesh = plsc.VectorSubcoreMesh(core_axis_name="core",
                                  subcore_axis_name="subcore")

    @pl.kernel(out_type=jax.ShapeDtypeStruct((batch, _SEQP, dim),
                                             embedding_table.dtype),
               mesh=mesh,
               scratch_types=[pltpu.SemaphoreType.DMA],
               compiler_params=pltpu.CompilerParams(use_tc_tiling_on_sc=False))
    def gather_kernel(table_hbm, i_hbm, o_hbm, sem):
        def body(i_vmem, o_vmem):
            copies = [
                pltpu.async_copy(table_hbm.at[i_vmem.at[j, pl.ds(0, _SEQP)]],
                                 o_vmem.at[j], sem)
                for j in range(_R)
            ]
            for c in copies:
                c.wait()

        pltpu.emit_pipeline(
            body,
            grid=(batch // _R,),
            in_specs=[pl.BlockSpec((_R, 128), lambda i: (i, 0))],
            out_specs=[pl.BlockSpec((_R, _SEQP, dim), lambda i: (i, 0, 0))],
            core_axis_name=("core", "subcore"),
            dimension_semantics=(pltpu.PARALLEL,),
        )(i_hbm, o_hbm)

    out = gather_kernel(embedding_table, ids_pad)
    embeddings = out[:, :seq, :]
    attention_mask = jnp.ones((batch, seq), dtype=jnp.int32)
    return (embeddings, attention_mask)


# speedup vs baseline: 1.3963x; 1.3963x over previous
import jax
import jax.numpy as jnp
from jax.experimental import pallas as pl
from jax.experimental.pallas import tpu as pltpu
from jax.experimental.pallas import tpu_sc as plsc

_R = 8


def kernel(input_ids, embedding_table):
    batch, seq = input_ids.shape
    vocab, dim = embedding_table.shape
    table2 = embedding_table.reshape(vocab // 2, 2 * dim)
    half_ids = input_ids >> 1

    mesh = plsc.VectorSubcoreMesh(core_axis_name="core",
                                  subcore_axis_name="subcore")

    @pl.kernel(out_type=jax.ShapeDtypeStruct((batch, seq, 2 * dim),
                                             embedding_table.dtype),
               mesh=mesh,
               scratch_types=[pltpu.SemaphoreType.DMA],
               compiler_params=pltpu.CompilerParams(use_tc_tiling_on_sc=False))
    def gather_kernel(table_hbm, i_hbm, o_hbm, sem):
        def body(i_vmem, o_vmem):
            copies = [
                pltpu.async_copy(table_hbm.at[i_vmem.at[j]],
                                 o_vmem.at[j], sem)
                for j in range(_R)
            ]
            for c in copies:
                c.wait()

        pltpu.emit_pipeline(
            body,
            grid=(batch // _R,),
            in_specs=[pl.BlockSpec((_R, seq), lambda i: (i, 0))],
            out_specs=[pl.BlockSpec((_R, seq, 2 * dim), lambda i: (i, 0, 0))],
            core_axis_name=("core", "subcore"),
            dimension_semantics=(pltpu.PARALLEL,),
        )(i_hbm, o_hbm)

    pairs = gather_kernel(table2, half_ids)
    odd = (input_ids % 2).astype(jnp.bool_)[:, :, None]
    embeddings = jnp.where(odd, pairs[:, :, dim:], pairs[:, :, :dim])
    attention_mask = jnp.ones((batch, seq), dtype=jnp.int32)
    return (embeddings, attention_mask)


# speedup vs baseline: 1.7194x; 1.2314x over previous
import jax
import jax.numpy as jnp
from jax.experimental import pallas as pl
from jax.experimental.pallas import tpu as pltpu
from jax.experimental.pallas import tpu_sc as plsc

_R = 8


def kernel(input_ids, embedding_table):
    batch, seq = input_ids.shape
    vocab, dim = embedding_table.shape
    table_pad = jnp.pad(embedding_table, ((0, 0), (0, 128 - dim)))

    mesh = plsc.VectorSubcoreMesh(core_axis_name="core",
                                  subcore_axis_name="subcore")

    @pl.kernel(out_type=jax.ShapeDtypeStruct((batch, seq, 128),
                                             embedding_table.dtype),
               mesh=mesh,
               scratch_types=[pltpu.SemaphoreType.DMA],
               compiler_params=pltpu.CompilerParams(use_tc_tiling_on_sc=False))
    def gather_kernel(table_hbm, i_hbm, o_hbm, sem):
        def body(i_vmem, o_vmem):
            copies = [
                pltpu.async_copy(table_hbm.at[i_vmem.at[j]],
                                 o_vmem.at[j], sem)
                for j in range(_R)
            ]
            for c in copies:
                c.wait()

        pltpu.emit_pipeline(
            body,
            grid=(batch // _R,),
            in_specs=[pl.BlockSpec((_R, seq), lambda i: (i, 0))],
            out_specs=[pl.BlockSpec((_R, seq, 128), lambda i: (i, 0, 0))],
            core_axis_name=("core", "subcore"),
            dimension_semantics=(pltpu.PARALLEL,),
        )(i_hbm, o_hbm)

    padded_rows = gather_kernel(table_pad, input_ids)
    embeddings = padded_rows[:, :, :dim]
    attention_mask = jnp.ones((batch, seq), dtype=jnp.int32)
    return (embeddings, attention_mask)
